# baseline (speedup 1.0000x reference)
import functools

import jax
import jax.numpy as jnp
from jax import lax
from jax.experimental import pallas as pl
from jax.experimental.pallas import tpu as pltpu
from jax.experimental.pallas import tpu_sc as plsc

B, C, N = 2, 6, 24000
K = 10
S = 3072
H = 256

NP = 24576
BLK1 = 3072
TW = 128

NW = 32
TOTAL = B * K * S
PER_W = TOTAL // NW
CHUNK = 128
NCH = PER_W // CHUNK


def _stage1_body(pts_ref, w1_ref, b1_ref, wh_ref, head_ref, ptst_ref):
    x = pts_ref[0]
    xt = x.T
    feat = jax.nn.relu(
        jax.lax.dot_general(xt, w1_ref[...], (((1,), (0,)), ((), ())),
                            preferred_element_type=jnp.float32)
        + b1_ref[...][None, :]
    )
    head = jax.lax.dot_general(feat, wh_ref[...], (((1,), (0,)), ((), ())),
                               preferred_element_type=jnp.float32)
    head_ref[0] = head
    ptst_ref[0] = jnp.pad(xt, ((0, 0), (0, TW - C)))


def _stage1(points, W1, b1, Whead):
    points = jnp.pad(points, ((0, 0), (0, 0), (0, NP - N)))
    nb = NP // BLK1
    head, pts_t = pl.pallas_call(
        _stage1_body,
        grid=(B, nb),
        in_specs=[
            pl.BlockSpec((1, C, BLK1), lambda b, j: (b, 0, j)),
            pl.BlockSpec((C, H), lambda b, j: (0, 0)),
            pl.BlockSpec((H,), lambda b, j: (0,)),
            pl.BlockSpec((H, 16), lambda b, j: (0, 0)),
        ],
        out_specs=[
            pl.BlockSpec((1, BLK1, 16), lambda b, j: (b, j, 0)),
            pl.BlockSpec((1, BLK1, TW), lambda b, j: (b, j, 0)),
        ],
        out_shape=[
            jax.ShapeDtypeStruct((B, NP, 16), jnp.float32),
            jax.ShapeDtypeStruct((B, NP, TW), jnp.float32),
        ],
    )(points, W1, b1, Whead)
    return head[:, :N], pts_t


def _sc_gather(table, idx3):
    mesh = plsc.VectorSubcoreMesh(core_axis_name="c", subcore_axis_name="s")

    @functools.partial(
        pl.kernel, mesh=mesh,
        out_type=jax.ShapeDtypeStruct((NW, NCH, CHUNK, TW), jnp.float32),
        scratch_types=[
            pltpu.VMEM((NCH, CHUNK), jnp.int32),
            pltpu.VMEM((CHUNK, TW), jnp.float32),
            pltpu.SemaphoreType.DMA,
        ],
    )
    def gk(table_hbm, idx_hbm, out_hbm, idx_v, rows_v, sem):
        wid = lax.axis_index("s") * 2 + lax.axis_index("c")
        pltpu.sync_copy(idx_hbm.at[wid], idx_v)
        for j in range(NCH):
            pltpu.async_copy(table_hbm.at[idx_v.at[j]], rows_v, sem).wait()
            pltpu.sync_copy(rows_v, out_hbm.at[wid].at[j])

    return gk(table, idx3)


def _stage2_body(crop_ref, w2_ref, b2_ref, wh_ref, head_ref, cent_ref):
    xt = crop_ref[0, :, :C]
    xyz = xt[:, :3]
    mean = jnp.sum(xyz, axis=0, keepdims=True) / S
    ctr = jnp.concatenate([xyz - mean, xt[:, 3:]], axis=1)
    feat = jax.nn.relu(
        jax.lax.dot_general(ctr, w2_ref[...], (((1,), (0,)), ((), ())),
                            preferred_element_type=jnp.float32)
        + b2_ref[...][None, :]
    )
    head = jax.lax.dot_general(feat, wh_ref[...], (((1,), (0,)), ((), ())),
                               preferred_element_type=jnp.float32)
    head_ref[0] = head
    cent_ref[0] = ctr.T


def _stage2(cropped_t, W2, b2, Whead):
    head, centered = pl.pallas_call(
        _stage2_body,
        grid=(B * K,),
        in_specs=[
            pl.BlockSpec((1, S, TW), lambda i: (i, 0, 0)),
            pl.BlockSpec((C, H), lambda i: (0, 0)),
            pl.BlockSpec((H,), lambda i: (0,)),
            pl.BlockSpec((H, 8), lambda i: (0, 0)),
        ],
        out_specs=[
            pl.BlockSpec((1, S, 8), lambda i: (i, 0, 0)),
            pl.BlockSpec((1, C, S), lambda i: (i, 0, 0)),
        ],
        out_shape=[
            jax.ShapeDtypeStruct((B * K, S, 8), jnp.float32),
            jax.ShapeDtypeStruct((B * K, C, S), jnp.float32),
        ],
    )(cropped_t, W2, b2, Whead)
    return head, centered


def kernel(points, labels, W1, b1, Wsem1, Woff1, Wmask1, W2, b2, Wsem2, Woff2, Wmask2):
    Whead1 = jnp.pad(jnp.concatenate([Wsem1, Woff1, Wmask1], axis=1),
                     ((0, 0), (0, 2)))
    Whead2 = jnp.pad(jnp.concatenate([Wsem2, Woff2, Wmask2], axis=1),
                     ((0, 0), (0, 2)))

    head1, pts_t = _stage1(points, W1, b1, Whead1)
    sem1 = head1[:, :, :K]
    off1 = head1[:, :, K:K + 3]
    mask1 = head1[:, :, K + 3:K + 4]

    coords = jnp.swapaxes(points[:, :3, :], 1, 2)
    lab = labels[:, 0, :]

    def centroids_b(cb, lb):
        s = jax.ops.segment_sum(cb, lb, num_segments=K)
        cnt = jax.ops.segment_sum(jnp.ones((cb.shape[0],), jnp.float32), lb,
                                  num_segments=K)
        return s / jnp.maximum(cnt, 1.0)[:, None]

    cents = jax.vmap(centroids_b)(coords, lab)

    d2 = jnp.sum((coords[:, None, :, :] - cents[:, :, None, :]) ** 2, axis=-1)
    keys = jax.lax.bitcast_convert_type(d2, jnp.int32)
    _, idx = jax.lax.top_k(-keys, S)

    flat_idx = (idx + (jnp.arange(B, dtype=idx.dtype) * NP)[:, None, None]
                ).reshape(NW, NCH, CHUNK)
    rows = _sc_gather(pts_t.reshape(B * NP, TW), flat_idx)
    cropped_t = rows.reshape(B * K, S, TW)

    head2, centered = _stage2(cropped_t, W2, b2, Whead2)
    sem2 = head2[:, :, :2]
    off2 = head2[:, :, 2:5]
    mask2 = head2[:, :, 5:6]
    centered = centered.reshape(B, K, C, S)
    return (sem1, off1, mask1, sem2, off2, mask2, centered)
